# baseline (speedup 1.0000x reference)
import functools

import numpy as np
import jax
import jax.numpy as jnp
from jax import lax
from jax.experimental import pallas as pl
from jax.experimental.pallas import tpu as pltpu
from jax.experimental.pallas import tpu_sc as plsc

C, T, V = 64, 2048, 128
LOW, HIGH = 512, 1024

_rng = np.random.default_rng(0)
_num = int(_rng.integers(LOW, HIGH + 1))
_masked_inds = np.asarray(_rng.choice(T, _num, replace=False), dtype=np.int64)

NC, NS = 2, 16
NW = NC * NS
FPW = T // NW
ZPAD = 128


def _build_masked_table():
    tab = np.full((NW, ZPAD), _masked_inds[0], dtype=np.int32)
    for w in range(NW):
        lo, hi = w * FPW, (w + 1) * FPW
        mine = _masked_inds[(_masked_inds >= lo) & (_masked_inds < hi)]
        tab[w, : len(mine)] = np.sort(mine)
    return tab


_MTAB = _build_masked_table()

_mesh = plsc.VectorSubcoreMesh(core_axis_name="c", subcore_axis_name="s")


@functools.partial(
    pl.kernel,
    mesh=_mesh,
    out_type=jax.ShapeDtypeStruct((T, V), jnp.float32),
    scratch_types=[
        pltpu.VMEM((ZPAD,), jnp.int32),
        pltpu.VMEM((FPW, V), jnp.float32),
        pltpu.VMEM((ZPAD, V), jnp.float32),
        pltpu.SemaphoreType.DMA,
    ],
)
def _sc_make_fm(ones_hbm, z_hbm, mt_hbm, fm_hbm, mtv, onesv, zv, zsem):
    wid = lax.axis_index("s") * NC + lax.axis_index("c")
    pltpu.sync_copy(mt_hbm.at[wid], mtv)
    pltpu.sync_copy(ones_hbm, onesv)
    pltpu.sync_copy(z_hbm, zv)
    pltpu.sync_copy(onesv, fm_hbm.at[pl.ds(wid * FPW, FPW)])
    pltpu.async_copy(zv, fm_hbm.at[mtv], zsem).wait()


_BR = 8192


def _mul_body(mask_ref, fm_ref, out_ref, fm_vmem):
    @pl.when(pl.program_id(0) == 0)
    def _init():
        for k in range(_BR // T):
            fm_vmem[pl.ds(k * T, T), :] = fm_ref[...]

    out_ref[...] = mask_ref[...] * fm_vmem[...]


def kernel(x, mask):
    ones = jnp.ones((FPW, V), jnp.float32)
    zeros = jnp.zeros((ZPAD, V), jnp.float32)
    fm2d = _sc_make_fm(ones, zeros, jnp.asarray(_MTAB))
    m2d = mask.reshape(C * T, V)
    out = pl.pallas_call(
        _mul_body,
        grid=(C * T // _BR,),
        in_specs=[
            pl.BlockSpec((_BR, V), lambda i: (i, 0)),
            pl.BlockSpec((T, V), lambda i: (0, 0)),
        ],
        out_specs=pl.BlockSpec((_BR, V), lambda i: (i, 0)),
        out_shape=jax.ShapeDtypeStruct((C * T, V), jnp.float32),
        scratch_shapes=[pltpu.VMEM((_BR, V), jnp.float32)],
    )(m2d, fm2d)
    return (x, out.reshape(C, T, V))


# speedup vs baseline: 2.0572x; 2.0572x over previous
import functools

import numpy as np
import jax
import jax.numpy as jnp
from jax import lax
from jax.experimental import pallas as pl
from jax.experimental.pallas import tpu as pltpu
from jax.experimental.pallas import tpu_sc as plsc

C, T, V = 64, 2048, 128
LOW, HIGH = 512, 1024

_rng = np.random.default_rng(0)
_num = int(_rng.integers(LOW, HIGH + 1))
_masked_inds = np.asarray(_rng.choice(T, _num, replace=False), dtype=np.int64)

NC, NS = 2, 16
NW = NC * NS
FPW = T // NW
ZPAD = 64


def _build_masked_table():
    tab = np.zeros((NW, ZPAD), dtype=np.int32)
    for w in range(NW):
        lo, hi = w * FPW, (w + 1) * FPW
        mine = np.sort(_masked_inds[(_masked_inds >= lo) & (_masked_inds < hi)])
        if len(mine) == 0:
            mine = _masked_inds[w : w + 1]
        tab[w] = np.resize(mine, ZPAD)
    return tab


_MTAB = _build_masked_table()

_mesh = plsc.VectorSubcoreMesh(core_axis_name="c", subcore_axis_name="s")


@functools.partial(
    pl.kernel,
    mesh=_mesh,
    out_type=jax.ShapeDtypeStruct((T, V), jnp.float32),
    scratch_types=[
        pltpu.VMEM((ZPAD,), jnp.int32),
        pltpu.VMEM((FPW, V), jnp.float32),
        pltpu.VMEM((ZPAD, V), jnp.float32),
        pltpu.SemaphoreType.DMA,
    ],
)
def _sc_make_fm(ones_hbm, z_hbm, mt_hbm, fm_hbm, mtv, onesv, zv, zsem):
    wid = lax.axis_index("s") * NC + lax.axis_index("c")
    pltpu.sync_copy(mt_hbm.at[wid], mtv)
    pltpu.sync_copy(ones_hbm, onesv)
    pltpu.sync_copy(z_hbm, zv)
    pltpu.sync_copy(onesv, fm_hbm.at[pl.ds(wid * FPW, FPW)])
    pltpu.async_copy(zv, fm_hbm.at[mtv], zsem).wait()


_BR = 8192


def _mul_body(mask_ref, fm_ref, out_ref, fm_vmem):
    @pl.when(pl.program_id(0) == 0)
    def _init():
        for k in range(_BR // T):
            fm_vmem[pl.ds(k * T, T), :] = fm_ref[...]

    out_ref[...] = mask_ref[...] * fm_vmem[...]


def kernel(x, mask):
    ones = jnp.ones((FPW, V), jnp.float32)
    zeros = jnp.zeros((ZPAD, V), jnp.float32)
    fm2d = _sc_make_fm(ones, zeros, jnp.asarray(_MTAB))
    m2d = mask.reshape(C * T, V)
    out = pl.pallas_call(
        _mul_body,
        grid=(C * T // _BR,),
        in_specs=[
            pl.BlockSpec((_BR, V), lambda i: (i, 0)),
            pl.BlockSpec((T, V), lambda i: (0, 0)),
        ],
        out_specs=pl.BlockSpec((_BR, V), lambda i: (i, 0)),
        out_shape=jax.ShapeDtypeStruct((C * T, V), jnp.float32),
        scratch_shapes=[pltpu.VMEM((_BR, V), jnp.float32)],
    )(m2d, fm2d)
    return (x, out.reshape(C, T, V))


# speedup vs baseline: 2.0763x; 1.0093x over previous
import functools

import numpy as np
import jax
import jax.numpy as jnp
from jax import lax
from jax.experimental import pallas as pl
from jax.experimental.pallas import tpu as pltpu
from jax.experimental.pallas import tpu_sc as plsc

C, T, V = 64, 2048, 128
LOW, HIGH = 512, 1024

_rng = np.random.default_rng(0)
_num = int(_rng.integers(LOW, HIGH + 1))
_masked_inds = np.asarray(_rng.choice(T, _num, replace=False), dtype=np.int64)

NC, NS = 2, 16
NW = NC * NS
FPW = T // NW
ZPAD = 64


def _build_masked_table():
    tab = np.zeros((NW, ZPAD), dtype=np.int32)
    for w in range(NW):
        lo, hi = w * FPW, (w + 1) * FPW
        mine = np.sort(_masked_inds[(_masked_inds >= lo) & (_masked_inds < hi)])
        if len(mine) == 0:
            mine = _masked_inds[w : w + 1]
        tab[w] = np.resize(mine, ZPAD)
    return tab


_MTAB = _build_masked_table()

_mesh = plsc.VectorSubcoreMesh(core_axis_name="c", subcore_axis_name="s")


@functools.partial(
    pl.kernel,
    mesh=_mesh,
    out_type=jax.ShapeDtypeStruct((T, V), jnp.float32),
    scratch_types=[
        pltpu.VMEM((ZPAD,), jnp.int32),
        pltpu.VMEM((FPW, V), jnp.float32),
        pltpu.VMEM((ZPAD, V), jnp.float32),
        pltpu.SemaphoreType.DMA,
        pltpu.SemaphoreType.DMA,
        pltpu.SemaphoreType.DMA,
    ],
)
def _sc_make_fm(ones_hbm, z_hbm, mt_hbm, fm_hbm, mtv, onesv, zv,
                sem0, sem1, sem2):
    wid = lax.axis_index("s") * NC + lax.axis_index("c")
    h0 = pltpu.async_copy(mt_hbm.at[wid], mtv, sem0)
    h1 = pltpu.async_copy(ones_hbm, onesv, sem1)
    h2 = pltpu.async_copy(z_hbm, zv, sem2)
    h1.wait()
    pltpu.sync_copy(onesv, fm_hbm.at[pl.ds(wid * FPW, FPW)])
    h0.wait()
    h2.wait()
    pltpu.async_copy(zv, fm_hbm.at[mtv], sem1).wait()


_BR = 8192


def _mul_body(mask_ref, fm_ref, out_ref, fm_vmem):
    @pl.when(pl.program_id(0) == 0)
    def _init():
        for k in range(_BR // T):
            fm_vmem[pl.ds(k * T, T), :] = fm_ref[...]

    out_ref[...] = mask_ref[...] * fm_vmem[...]


def kernel(x, mask):
    ones = jnp.ones((FPW, V), jnp.float32)
    zeros = jnp.zeros((ZPAD, V), jnp.float32)
    fm2d = _sc_make_fm(ones, zeros, jnp.asarray(_MTAB))
    m2d = mask.reshape(C * T, V)
    out = pl.pallas_call(
        _mul_body,
        grid=(C * T // _BR,),
        in_specs=[
            pl.BlockSpec((_BR, V), lambda i: (i, 0)),
            pl.BlockSpec((T, V), lambda i: (0, 0)),
        ],
        out_specs=pl.BlockSpec((_BR, V), lambda i: (i, 0)),
        out_shape=jax.ShapeDtypeStruct((C * T, V), jnp.float32),
        scratch_shapes=[pltpu.VMEM((_BR, V), jnp.float32)],
    )(m2d, fm2d)
    return (x, out.reshape(C, T, V))
